# Initial kernel scaffold; baseline (speedup 1.0000x reference)
import jax
import jax.numpy as jnp
from jax.experimental import pallas as pl


def kernel(x, o_adj, s_adj, idx, W_ogc1, b_ogc1, W_ogc2, b_ogc2, W_ogc1s, b_ogc1s, W_sgc1, b_sgc1, W_sgc2, b_sgc2, W_sgc1o, b_sgc1o, W_sgc2o, b_sgc2o, gate_o1, gate_s1, gate_o2, W_dec1, b_dec1, W_dec2, b_dec2):
    raise NotImplementedError("write your pallas kernel here")


# speedup vs baseline: 1.4617x; 1.4617x over previous
import functools

import jax
import jax.numpy as jnp
from jax import lax
from jax.experimental import pallas as pl
from jax.experimental.pallas import tpu as pltpu
from jax.experimental.pallas import tpu_sc as plsc


def _relu(v):
    return jnp.maximum(v, 0.0)


def _rowblk(n):
    return 80 if n % 80 == 0 else n


def _proj(x, w):
    n, f = x.shape
    fo = w.shape[1]
    m = 2000 if n % 2000 == 0 else n

    def body(x_ref, w_ref, o_ref):
        o_ref[...] = jnp.dot(x_ref[...], w_ref[...],
                             preferred_element_type=jnp.float32)

    return pl.pallas_call(
        body,
        grid=(n // m,),
        in_specs=[
            pl.BlockSpec((m, f), lambda i: (i, 0)),
            pl.BlockSpec((f, fo), lambda i: (0, 0)),
        ],
        out_specs=pl.BlockSpec((m, fo), lambda i: (i, 0)),
        out_shape=jax.ShapeDtypeStruct((n, fo), jnp.float32),
    )(x, w)


def _pass1(o_adj, s_adj, s_all, g_o1, b_o1, b_s1o, b_s1, w_o1s, w_o2):
    n = o_adj.shape[0]
    m = _rowblk(n)

    def body(oa_ref, sa_ref, s_ref, go1_ref, bo1_ref, bs1o_ref, bs1_ref,
             w1s_ref, w2_ref, r2_ref, ca_ref, x2a_ref):
        a = jnp.dot(oa_ref[...], s_ref[:, 0:64],
                    preferred_element_type=jnp.float32)
        bc = jnp.dot(sa_ref[...], s_ref[:, 64:192],
                     preferred_element_type=jnp.float32)
        bv = bc[:, 0:64]
        cv = bc[:, 64:128]
        go1 = go1_ref[...]
        apb = a + bo1_ref[...]
        o_x = _relu(go1 * apb + (1.0 - go1) * (bv + bs1o_ref[...]))
        x1a = _relu(apb)
        ca = cv + bs1_ref[...]
        ca_ref[...] = ca
        x2a_ref[...] = _relu(ca)
        r2_ref[...] = jnp.concatenate(
            [jnp.dot(o_x, w1s_ref[...], preferred_element_type=jnp.float32),
             jnp.dot(o_x, w2_ref[...], preferred_element_type=jnp.float32),
             jnp.dot(x1a, w2_ref[...], preferred_element_type=jnp.float32)],
            axis=1)

    const = lambda i: (0, 0)
    return pl.pallas_call(
        body,
        grid=(n // m,),
        in_specs=[
            pl.BlockSpec((m, n), lambda i: (i, 0)),
            pl.BlockSpec((m, n), lambda i: (i, 0)),
            pl.BlockSpec((n, 192), const),
            pl.BlockSpec((1, 64), const),
            pl.BlockSpec((1, 64), const),
            pl.BlockSpec((1, 64), const),
            pl.BlockSpec((1, 64), const),
            pl.BlockSpec((64, 64), const),
            pl.BlockSpec((64, 32), const),
        ],
        out_specs=[
            pl.BlockSpec((m, 128), lambda i: (i, 0)),
            pl.BlockSpec((m, 64), lambda i: (i, 0)),
            pl.BlockSpec((m, 64), lambda i: (i, 0)),
        ],
        out_shape=[
            jax.ShapeDtypeStruct((n, 128), jnp.float32),
            jax.ShapeDtypeStruct((n, 64), jnp.float32),
            jax.ShapeDtypeStruct((n, 64), jnp.float32),
        ],
        compiler_params=pltpu.CompilerParams(
            dimension_semantics=("parallel",),
            vmem_limit_bytes=100 * 1024 * 1024,
        ),
    )(o_adj, s_adj, s_all, g_o1, b_o1, b_s1o, b_s1, w_o1s, w_o2)


def _pass2(o_adj, r2, ca, x2a, g_s1, b_o1s, b_o2, w_s2o, w_s2):
    n = o_adj.shape[0]
    m = _rowblk(n)

    def body(oa_ref, r2_ref, ca_ref, x2a_ref, gs1_ref, b1s_ref, b2_ref,
             ws2o_ref, ws2_ref, r3_ref, x1_ref, ea_ref):
        acc = jnp.dot(oa_ref[...], r2_ref[...],
                      preferred_element_type=jnp.float32)
        d = acc[:, 0:64]
        e = acc[:, 64:96]
        g = acc[:, 96:128]
        gs1 = gs1_ref[...]
        s_x = _relu(gs1 * ca_ref[...] + (1.0 - gs1) * (d + b1s_ref[...]))
        x1_ref[...] = g + b2_ref[...]
        ea_ref[...] = e + b2_ref[...]
        r3_ref[...] = jnp.concatenate(
            [jnp.dot(s_x, ws2o_ref[...], preferred_element_type=jnp.float32),
             jnp.dot(x2a_ref[...], ws2_ref[...],
                     preferred_element_type=jnp.float32)],
            axis=1)

    const = lambda i: (0, 0)
    return pl.pallas_call(
        body,
        grid=(n // m,),
        in_specs=[
            pl.BlockSpec((m, n), lambda i: (i, 0)),
            pl.BlockSpec((n, 128), const),
            pl.BlockSpec((m, 64), lambda i: (i, 0)),
            pl.BlockSpec((m, 64), lambda i: (i, 0)),
            pl.BlockSpec((1, 64), const),
            pl.BlockSpec((1, 64), const),
            pl.BlockSpec((1, 32), const),
            pl.BlockSpec((64, 32), const),
            pl.BlockSpec((64, 32), const),
        ],
        out_specs=[
            pl.BlockSpec((m, 64), lambda i: (i, 0)),
            pl.BlockSpec((m, 32), lambda i: (i, 0)),
            pl.BlockSpec((m, 32), lambda i: (i, 0)),
        ],
        out_shape=[
            jax.ShapeDtypeStruct((n, 64), jnp.float32),
            jax.ShapeDtypeStruct((n, 32), jnp.float32),
            jax.ShapeDtypeStruct((n, 32), jnp.float32),
        ],
        compiler_params=pltpu.CompilerParams(
            dimension_semantics=("parallel",),
            vmem_limit_bytes=100 * 1024 * 1024,
        ),
    )(o_adj, r2, ca, x2a, g_s1, b_o1s, b_o2, w_s2o, w_s2)


def _pass3(s_adj, r3, x1, ea, g_o2, b_s2o, b_s2, uvcat, chalf):
    n = s_adj.shape[0]
    m = _rowblk(n)

    def body(sa_ref, r3_ref, x1_ref, ea_ref, go2_ref, bs2o_ref, bs2_ref,
             uv_ref, c_ref, xall_ref, pq_ref):
        acc = jnp.dot(sa_ref[...], r3_ref[...],
                      preferred_element_type=jnp.float32)
        f = acc[:, 0:32]
        h = acc[:, 32:64]
        go2 = go2_ref[...]
        x_feat = go2 * ea_ref[...] + (1.0 - go2) * (f + bs2o_ref[...])
        x_2 = h + bs2_ref[...]
        x_all = jnp.concatenate([x1_ref[...], x_2, x_feat], axis=1)
        xall_ref[...] = x_all
        pq_ref[...] = jnp.dot(x_all, uv_ref[...],
                              preferred_element_type=jnp.float32) + c_ref[...]

    const = lambda i: (0, 0)
    return pl.pallas_call(
        body,
        grid=(n // m,),
        in_specs=[
            pl.BlockSpec((m, n), lambda i: (i, 0)),
            pl.BlockSpec((n, 64), const),
            pl.BlockSpec((m, 32), lambda i: (i, 0)),
            pl.BlockSpec((m, 32), lambda i: (i, 0)),
            pl.BlockSpec((1, 32), const),
            pl.BlockSpec((1, 32), const),
            pl.BlockSpec((1, 32), const),
            pl.BlockSpec((96, 2), const),
            pl.BlockSpec((1, 1), const),
        ],
        out_specs=[
            pl.BlockSpec((m, 96), lambda i: (i, 0)),
            pl.BlockSpec((m, 2), lambda i: (i, 0)),
        ],
        out_shape=[
            jax.ShapeDtypeStruct((n, 96), jnp.float32),
            jax.ShapeDtypeStruct((n, 2), jnp.float32),
        ],
        compiler_params=pltpu.CompilerParams(
            dimension_semantics=("parallel",),
            vmem_limit_bytes=100 * 1024 * 1024,
        ),
    )(s_adj, r3, x1, ea, g_o2, b_s2o, b_s2, uvcat, chalf)


def _tc_forward(x, o_adj, s_adj, W_ogc1, b_ogc1, W_ogc2, b_ogc2, W_ogc1s,
                b_ogc1s, W_sgc1, b_sgc1, W_sgc2, b_sgc2, W_sgc1o, b_sgc1o,
                W_sgc2o, b_sgc2o, gate_o1, gate_s1, gate_o2, W_dec1, b_dec1,
                W_dec2, b_dec2):
    row = lambda v: v.reshape(1, -1)
    s_all = _proj(x, jnp.concatenate([W_ogc1, W_sgc1o, W_sgc1], axis=1))
    r2, ca, x2a = _pass1(o_adj, s_adj, s_all, row(gate_o1), row(b_ogc1),
                         row(b_sgc1o), row(b_sgc1), W_ogc1s, W_ogc2)
    r3, x1, ea = _pass2(o_adj, r2, ca, x2a, row(gate_s1), row(b_ogc1s),
                        row(b_ogc2), W_sgc2o, W_sgc2)
    nh = W_dec1.shape[0] // 2
    uv = W_dec1 @ W_dec2
    uvcat = jnp.concatenate([uv[:nh], uv[nh:]], axis=1)
    c = b_dec1 @ W_dec2 + b_dec2
    x_all, pq = _pass3(s_adj, r3, x1, ea, row(gate_o2), row(b_sgc2o),
                       row(b_sgc2), uvcat, (0.5 * c).reshape(1, 1))
    return x_all, pq


def _sc_linkpred(p, q, i0, i1):
    n, = p.shape
    b, = i0.shape
    info = plsc.get_sparse_core_info()
    nc, ns, lanes = info.num_cores, info.num_subcores, info.num_lanes
    nw = nc * ns
    bpw = b // nw
    mesh = plsc.VectorSubcoreMesh(core_axis_name="c", subcore_axis_name="s")

    @functools.partial(
        pl.kernel,
        mesh=mesh,
        out_type=jax.ShapeDtypeStruct((b,), jnp.float32),
        compiler_params=pltpu.CompilerParams(needs_layout_passes=False),
        scratch_types=[
            pltpu.VMEM((n,), jnp.float32),
            pltpu.VMEM((n,), jnp.float32),
            pltpu.VMEM((bpw,), jnp.int32),
            pltpu.VMEM((bpw,), jnp.int32),
            pltpu.VMEM((bpw,), jnp.float32),
        ],
    )
    def k(p_hbm, q_hbm, i0_hbm, i1_hbm, out_hbm, p_v, q_v, i0_v, i1_v, o_v):
        wid = lax.axis_index("s") * nc + lax.axis_index("c")
        base = wid * bpw
        pltpu.sync_copy(p_hbm, p_v)
        pltpu.sync_copy(q_hbm, q_v)
        pltpu.sync_copy(i0_hbm.at[pl.ds(base, bpw)], i0_v)
        pltpu.sync_copy(i1_hbm.at[pl.ds(base, bpw)], i1_v)

        def body(t, carry):
            sl = pl.ds(t * lanes, lanes)
            a = plsc.load_gather(p_v, [i0_v[sl]])
            bb = plsc.load_gather(q_v, [i1_v[sl]])
            o_v[sl] = a + bb
            return carry

        lax.fori_loop(0, bpw // lanes, body, 0)
        pltpu.sync_copy(o_v, out_hbm.at[pl.ds(base, bpw)])

    return k(p, q, i0, i1)


def kernel(x, o_adj, s_adj, idx, W_ogc1, b_ogc1, W_ogc2, b_ogc2, W_ogc1s,
           b_ogc1s, W_sgc1, b_sgc1, W_sgc2, b_sgc2, W_sgc1o, b_sgc1o, W_sgc2o,
           b_sgc2o, gate_o1, gate_s1, gate_o2, W_dec1, b_dec1, W_dec2, b_dec2):
    x_all, pq = _tc_forward(x, o_adj, s_adj, W_ogc1, b_ogc1, W_ogc2, b_ogc2,
                            W_ogc1s, b_ogc1s, W_sgc1, b_sgc1, W_sgc2, b_sgc2,
                            W_sgc1o, b_sgc1o, W_sgc2o, b_sgc2o, gate_o1,
                            gate_s1, gate_o2, W_dec1, b_dec1, W_dec2, b_dec2)
    o = _sc_linkpred(pq[:, 0], pq[:, 1], idx[0], idx[1])
    return o.reshape(-1, 1), x_all


# speedup vs baseline: 1.9552x; 1.3377x over previous
import functools

import jax
import jax.numpy as jnp
from jax import lax
from jax.experimental import pallas as pl
from jax.experimental.pallas import tpu as pltpu
from jax.experimental.pallas import tpu_sc as plsc


def _relu(v):
    return jnp.maximum(v, 0.0)


def _rowblk(n):
    return 200 if n % 200 == 0 else n


def _proj(x, w):
    n, f = x.shape
    fo = w.shape[1]
    m = 2000 if n % 2000 == 0 else n

    def body(x_ref, w_ref, o_ref):
        o_ref[...] = jnp.dot(x_ref[...], w_ref[...],
                             preferred_element_type=jnp.float32)

    return pl.pallas_call(
        body,
        grid=(n // m,),
        in_specs=[
            pl.BlockSpec((m, f), lambda i: (i, 0)),
            pl.BlockSpec((f, fo), lambda i: (0, 0)),
        ],
        out_specs=pl.BlockSpec((m, fo), lambda i: (i, 0)),
        out_shape=jax.ShapeDtypeStruct((n, fo), jnp.float32),
    )(x, w)


def _pass1(o_adj, s_adj, s_all, g_o1, b_o1, b_s1o, b_s1, w_o1s, w_o2):
    n = o_adj.shape[0]
    m = _rowblk(n)

    def body(oa_ref, sa_ref, s_ref, go1_ref, bo1_ref, bs1o_ref, bs1_ref,
             w1s_ref, w2_ref, r2_ref, ca_ref, x2a_ref):
        a = jnp.dot(oa_ref[...], s_ref[:, 0:64],
                    preferred_element_type=jnp.float32)
        bc = jnp.dot(sa_ref[...], s_ref[:, 64:192],
                     preferred_element_type=jnp.float32)
        bv = bc[:, 0:64]
        cv = bc[:, 64:128]
        go1 = go1_ref[...]
        apb = a + bo1_ref[...]
        o_x = _relu(go1 * apb + (1.0 - go1) * (bv + bs1o_ref[...]))
        x1a = _relu(apb)
        ca = cv + bs1_ref[...]
        ca_ref[...] = ca
        x2a_ref[...] = _relu(ca)
        r2_ref[...] = jnp.concatenate(
            [jnp.dot(o_x, w1s_ref[...], preferred_element_type=jnp.float32),
             jnp.dot(o_x, w2_ref[...], preferred_element_type=jnp.float32),
             jnp.dot(x1a, w2_ref[...], preferred_element_type=jnp.float32)],
            axis=1)

    const = lambda i: (0, 0)
    return pl.pallas_call(
        body,
        grid=(n // m,),
        in_specs=[
            pl.BlockSpec((m, n), lambda i: (i, 0)),
            pl.BlockSpec((m, n), lambda i: (i, 0)),
            pl.BlockSpec((n, 192), const),
            pl.BlockSpec((1, 64), const),
            pl.BlockSpec((1, 64), const),
            pl.BlockSpec((1, 64), const),
            pl.BlockSpec((1, 64), const),
            pl.BlockSpec((64, 64), const),
            pl.BlockSpec((64, 32), const),
        ],
        out_specs=[
            pl.BlockSpec((m, 128), lambda i: (i, 0)),
            pl.BlockSpec((m, 64), lambda i: (i, 0)),
            pl.BlockSpec((m, 64), lambda i: (i, 0)),
        ],
        out_shape=[
            jax.ShapeDtypeStruct((n, 128), jnp.float32),
            jax.ShapeDtypeStruct((n, 64), jnp.float32),
            jax.ShapeDtypeStruct((n, 64), jnp.float32),
        ],
        compiler_params=pltpu.CompilerParams(
            dimension_semantics=("parallel",),
            vmem_limit_bytes=100 * 1024 * 1024,
        ),
    )(o_adj, s_adj, s_all, g_o1, b_o1, b_s1o, b_s1, w_o1s, w_o2)


def _pass2(o_adj, r2, ca, x2a, g_s1, b_o1s, b_o2, w_s2o, w_s2):
    n = o_adj.shape[0]
    m = _rowblk(n)

    def body(oa_ref, r2_ref, ca_ref, x2a_ref, gs1_ref, b1s_ref, b2_ref,
             ws2o_ref, ws2_ref, r3_ref, x1_ref, ea_ref):
        acc = jnp.dot(oa_ref[...], r2_ref[...],
                      preferred_element_type=jnp.float32)
        d = acc[:, 0:64]
        e = acc[:, 64:96]
        g = acc[:, 96:128]
        gs1 = gs1_ref[...]
        s_x = _relu(gs1 * ca_ref[...] + (1.0 - gs1) * (d + b1s_ref[...]))
        x1_ref[...] = g + b2_ref[...]
        ea_ref[...] = e + b2_ref[...]
        r3_ref[...] = jnp.concatenate(
            [jnp.dot(s_x, ws2o_ref[...], preferred_element_type=jnp.float32),
             jnp.dot(x2a_ref[...], ws2_ref[...],
                     preferred_element_type=jnp.float32)],
            axis=1)

    const = lambda i: (0, 0)
    return pl.pallas_call(
        body,
        grid=(n // m,),
        in_specs=[
            pl.BlockSpec((m, n), lambda i: (i, 0)),
            pl.BlockSpec((n, 128), const),
            pl.BlockSpec((m, 64), lambda i: (i, 0)),
            pl.BlockSpec((m, 64), lambda i: (i, 0)),
            pl.BlockSpec((1, 64), const),
            pl.BlockSpec((1, 64), const),
            pl.BlockSpec((1, 32), const),
            pl.BlockSpec((64, 32), const),
            pl.BlockSpec((64, 32), const),
        ],
        out_specs=[
            pl.BlockSpec((m, 64), lambda i: (i, 0)),
            pl.BlockSpec((m, 32), lambda i: (i, 0)),
            pl.BlockSpec((m, 32), lambda i: (i, 0)),
        ],
        out_shape=[
            jax.ShapeDtypeStruct((n, 64), jnp.float32),
            jax.ShapeDtypeStruct((n, 32), jnp.float32),
            jax.ShapeDtypeStruct((n, 32), jnp.float32),
        ],
        compiler_params=pltpu.CompilerParams(
            dimension_semantics=("parallel",),
            vmem_limit_bytes=100 * 1024 * 1024,
        ),
    )(o_adj, r2, ca, x2a, g_s1, b_o1s, b_o2, w_s2o, w_s2)


def _pass3(s_adj, r3, x1, ea, g_o2, b_s2o, b_s2, uvcat, chalf):
    n = s_adj.shape[0]
    m = _rowblk(n)

    def body(sa_ref, r3_ref, x1_ref, ea_ref, go2_ref, bs2o_ref, bs2_ref,
             uv_ref, c_ref, xall_ref, pq_ref):
        acc = jnp.dot(sa_ref[...], r3_ref[...],
                      preferred_element_type=jnp.float32)
        f = acc[:, 0:32]
        h = acc[:, 32:64]
        go2 = go2_ref[...]
        x_feat = go2 * ea_ref[...] + (1.0 - go2) * (f + bs2o_ref[...])
        x_2 = h + bs2_ref[...]
        x_all = jnp.concatenate([x1_ref[...], x_2, x_feat], axis=1)
        xall_ref[...] = x_all
        pq_ref[...] = jnp.dot(x_all, uv_ref[...],
                              preferred_element_type=jnp.float32) + c_ref[...]

    const = lambda i: (0, 0)
    return pl.pallas_call(
        body,
        grid=(n // m,),
        in_specs=[
            pl.BlockSpec((m, n), lambda i: (i, 0)),
            pl.BlockSpec((n, 64), const),
            pl.BlockSpec((m, 32), lambda i: (i, 0)),
            pl.BlockSpec((m, 32), lambda i: (i, 0)),
            pl.BlockSpec((1, 32), const),
            pl.BlockSpec((1, 32), const),
            pl.BlockSpec((1, 32), const),
            pl.BlockSpec((96, 2), const),
            pl.BlockSpec((1, 1), const),
        ],
        out_specs=[
            pl.BlockSpec((m, 96), lambda i: (i, 0)),
            pl.BlockSpec((m, 2), lambda i: (i, 0)),
        ],
        out_shape=[
            jax.ShapeDtypeStruct((n, 96), jnp.float32),
            jax.ShapeDtypeStruct((n, 2), jnp.float32),
        ],
        compiler_params=pltpu.CompilerParams(
            dimension_semantics=("parallel",),
            vmem_limit_bytes=100 * 1024 * 1024,
        ),
    )(s_adj, r3, x1, ea, g_o2, b_s2o, b_s2, uvcat, chalf)


def _tc_forward(x, o_adj, s_adj, W_ogc1, b_ogc1, W_ogc2, b_ogc2, W_ogc1s,
                b_ogc1s, W_sgc1, b_sgc1, W_sgc2, b_sgc2, W_sgc1o, b_sgc1o,
                W_sgc2o, b_sgc2o, gate_o1, gate_s1, gate_o2, W_dec1, b_dec1,
                W_dec2, b_dec2):
    row = lambda v: v.reshape(1, -1)
    s_all = _proj(x, jnp.concatenate([W_ogc1, W_sgc1o, W_sgc1], axis=1))
    r2, ca, x2a = _pass1(o_adj, s_adj, s_all, row(gate_o1), row(b_ogc1),
                         row(b_sgc1o), row(b_sgc1), W_ogc1s, W_ogc2)
    r3, x1, ea = _pass2(o_adj, r2, ca, x2a, row(gate_s1), row(b_ogc1s),
                        row(b_ogc2), W_sgc2o, W_sgc2)
    nh = W_dec1.shape[0] // 2
    uv = W_dec1 @ W_dec2
    uvcat = jnp.concatenate([uv[:nh], uv[nh:]], axis=1)
    c = b_dec1 @ W_dec2 + b_dec2
    x_all, pq = _pass3(s_adj, r3, x1, ea, row(gate_o2), row(b_sgc2o),
                       row(b_sgc2), uvcat, (0.5 * c).reshape(1, 1))
    return x_all, pq


def _sc_linkpred(p, q, i0, i1):
    n, = p.shape
    b, = i0.shape
    info = plsc.get_sparse_core_info()
    nc, ns, lanes = info.num_cores, info.num_subcores, info.num_lanes
    nw = nc * ns
    bpw = b // nw
    mesh = plsc.VectorSubcoreMesh(core_axis_name="c", subcore_axis_name="s")

    @functools.partial(
        pl.kernel,
        mesh=mesh,
        out_type=jax.ShapeDtypeStruct((b,), jnp.float32),
        compiler_params=pltpu.CompilerParams(needs_layout_passes=False),
        scratch_types=[
            pltpu.VMEM((n,), jnp.float32),
            pltpu.VMEM((n,), jnp.float32),
            pltpu.VMEM((bpw,), jnp.int32),
            pltpu.VMEM((bpw,), jnp.int32),
            pltpu.VMEM((bpw,), jnp.float32),
        ],
    )
    def k(p_hbm, q_hbm, i0_hbm, i1_hbm, out_hbm, p_v, q_v, i0_v, i1_v, o_v):
        wid = lax.axis_index("s") * nc + lax.axis_index("c")
        base = wid * bpw
        pltpu.sync_copy(p_hbm, p_v)
        pltpu.sync_copy(q_hbm, q_v)
        pltpu.sync_copy(i0_hbm.at[pl.ds(base, bpw)], i0_v)
        pltpu.sync_copy(i1_hbm.at[pl.ds(base, bpw)], i1_v)

        def body(t, carry):
            sl = pl.ds(t * lanes, lanes)
            a = plsc.load_gather(p_v, [i0_v[sl]])
            bb = plsc.load_gather(q_v, [i1_v[sl]])
            o_v[sl] = a + bb
            return carry

        lax.fori_loop(0, bpw // lanes, body, 0)
        pltpu.sync_copy(o_v, out_hbm.at[pl.ds(base, bpw)])

    return k(p, q, i0, i1)


def kernel(x, o_adj, s_adj, idx, W_ogc1, b_ogc1, W_ogc2, b_ogc2, W_ogc1s,
           b_ogc1s, W_sgc1, b_sgc1, W_sgc2, b_sgc2, W_sgc1o, b_sgc1o, W_sgc2o,
           b_sgc2o, gate_o1, gate_s1, gate_o2, W_dec1, b_dec1, W_dec2, b_dec2):
    x_all, pq = _tc_forward(x, o_adj, s_adj, W_ogc1, b_ogc1, W_ogc2, b_ogc2,
                            W_ogc1s, b_ogc1s, W_sgc1, b_sgc1, W_sgc2, b_sgc2,
                            W_sgc1o, b_sgc1o, W_sgc2o, b_sgc2o, gate_o1,
                            gate_s1, gate_o2, W_dec1, b_dec1, W_dec2, b_dec2)
    o = _sc_linkpred(pq[:, 0], pq[:, 1], idx[0], idx[1])
    return o.reshape(-1, 1), x_all


# speedup vs baseline: 1.9696x; 1.0074x over previous
import functools

import jax
import jax.numpy as jnp
from jax import lax
from jax.experimental import pallas as pl
from jax.experimental.pallas import tpu as pltpu
from jax.experimental.pallas import tpu_sc as plsc


def _relu(v):
    return jnp.maximum(v, 0.0)


def _rowblk(n, m=200):
    return m if n % m == 0 else n


def _proj(x, w):
    n, f = x.shape
    fo = w.shape[1]
    m = 2000 if n % 2000 == 0 else n

    def body(x_ref, w_ref, o_ref):
        o_ref[...] = jnp.dot(x_ref[...], w_ref[...],
                             preferred_element_type=jnp.float32)

    return pl.pallas_call(
        body,
        grid=(n // m,),
        in_specs=[
            pl.BlockSpec((m, f), lambda i: (i, 0)),
            pl.BlockSpec((f, fo), lambda i: (0, 0)),
        ],
        out_specs=pl.BlockSpec((m, fo), lambda i: (i, 0)),
        out_shape=jax.ShapeDtypeStruct((n, fo), jnp.float32),
    )(x, w)


def _pass1(o_adj, s_adj, s_all, g_o1, b_o1, b_s1o, b_s1, w_o1s, w_o2):
    n = o_adj.shape[0]
    m = _rowblk(n)

    def body(oa_ref, sa_ref, s_ref, go1_ref, bo1_ref, bs1o_ref, bs1_ref,
             w1s_ref, w2_ref, r2_ref, ca_ref, x2a_ref):
        a = jnp.dot(oa_ref[...], s_ref[:, 0:64],
                    preferred_element_type=jnp.float32)
        bc = jnp.dot(sa_ref[...], s_ref[:, 64:192],
                     preferred_element_type=jnp.float32)
        bv = bc[:, 0:64]
        cv = bc[:, 64:128]
        go1 = go1_ref[...]
        apb = a + bo1_ref[...]
        o_x = _relu(go1 * apb + (1.0 - go1) * (bv + bs1o_ref[...]))
        x1a = _relu(apb)
        ca = cv + bs1_ref[...]
        ca_ref[...] = ca
        x2a_ref[...] = _relu(ca)
        r2_ref[...] = jnp.concatenate(
            [jnp.dot(o_x, w1s_ref[...], preferred_element_type=jnp.float32),
             jnp.dot(o_x, w2_ref[...], preferred_element_type=jnp.float32),
             jnp.dot(x1a, w2_ref[...], preferred_element_type=jnp.float32)],
            axis=1)

    const = lambda i: (0, 0)
    return pl.pallas_call(
        body,
        grid=(n // m,),
        in_specs=[
            pl.BlockSpec((m, n), lambda i: (i, 0)),
            pl.BlockSpec((m, n), lambda i: (i, 0)),
            pl.BlockSpec((n, 192), const),
            pl.BlockSpec((1, 64), const),
            pl.BlockSpec((1, 64), const),
            pl.BlockSpec((1, 64), const),
            pl.BlockSpec((1, 64), const),
            pl.BlockSpec((64, 64), const),
            pl.BlockSpec((64, 32), const),
        ],
        out_specs=[
            pl.BlockSpec((m, 128), lambda i: (i, 0)),
            pl.BlockSpec((m, 64), lambda i: (i, 0)),
            pl.BlockSpec((m, 64), lambda i: (i, 0)),
        ],
        out_shape=[
            jax.ShapeDtypeStruct((n, 128), jnp.float32),
            jax.ShapeDtypeStruct((n, 64), jnp.float32),
            jax.ShapeDtypeStruct((n, 64), jnp.float32),
        ],
        compiler_params=pltpu.CompilerParams(
            dimension_semantics=("parallel",),
            vmem_limit_bytes=100 * 1024 * 1024,
        ),
    )(o_adj, s_adj, s_all, g_o1, b_o1, b_s1o, b_s1, w_o1s, w_o2)


def _pass2(o_adj, r2, ca, x2a, g_s1, b_o1s, b_o2, w_s2o, w_s2):
    n = o_adj.shape[0]
    m = _rowblk(n, 400)

    def body(oa_ref, r2_ref, ca_ref, x2a_ref, gs1_ref, b1s_ref, b2_ref,
             ws2o_ref, ws2_ref, r3_ref, x1_ref, ea_ref):
        acc = jnp.dot(oa_ref[...], r2_ref[...],
                      preferred_element_type=jnp.float32)
        d = acc[:, 0:64]
        e = acc[:, 64:96]
        g = acc[:, 96:128]
        gs1 = gs1_ref[...]
        s_x = _relu(gs1 * ca_ref[...] + (1.0 - gs1) * (d + b1s_ref[...]))
        x1_ref[...] = g + b2_ref[...]
        ea_ref[...] = e + b2_ref[...]
        r3_ref[...] = jnp.concatenate(
            [jnp.dot(s_x, ws2o_ref[...], preferred_element_type=jnp.float32),
             jnp.dot(x2a_ref[...], ws2_ref[...],
                     preferred_element_type=jnp.float32)],
            axis=1)

    const = lambda i: (0, 0)
    return pl.pallas_call(
        body,
        grid=(n // m,),
        in_specs=[
            pl.BlockSpec((m, n), lambda i: (i, 0)),
            pl.BlockSpec((n, 128), const),
            pl.BlockSpec((m, 64), lambda i: (i, 0)),
            pl.BlockSpec((m, 64), lambda i: (i, 0)),
            pl.BlockSpec((1, 64), const),
            pl.BlockSpec((1, 64), const),
            pl.BlockSpec((1, 32), const),
            pl.BlockSpec((64, 32), const),
            pl.BlockSpec((64, 32), const),
        ],
        out_specs=[
            pl.BlockSpec((m, 64), lambda i: (i, 0)),
            pl.BlockSpec((m, 32), lambda i: (i, 0)),
            pl.BlockSpec((m, 32), lambda i: (i, 0)),
        ],
        out_shape=[
            jax.ShapeDtypeStruct((n, 64), jnp.float32),
            jax.ShapeDtypeStruct((n, 32), jnp.float32),
            jax.ShapeDtypeStruct((n, 32), jnp.float32),
        ],
        compiler_params=pltpu.CompilerParams(
            dimension_semantics=("parallel",),
            vmem_limit_bytes=100 * 1024 * 1024,
        ),
    )(o_adj, r2, ca, x2a, g_s1, b_o1s, b_o2, w_s2o, w_s2)


def _pass3(s_adj, r3, x1, ea, g_o2, b_s2o, b_s2, uvcat, chalf):
    n = s_adj.shape[0]
    m = _rowblk(n, 400)

    def body(sa_ref, r3_ref, x1_ref, ea_ref, go2_ref, bs2o_ref, bs2_ref,
             uv_ref, c_ref, xall_ref, pq_ref):
        acc = jnp.dot(sa_ref[...], r3_ref[...],
                      preferred_element_type=jnp.float32)
        f = acc[:, 0:32]
        h = acc[:, 32:64]
        go2 = go2_ref[...]
        x_feat = go2 * ea_ref[...] + (1.0 - go2) * (f + bs2o_ref[...])
        x_2 = h + bs2_ref[...]
        x_all = jnp.concatenate([x1_ref[...], x_2, x_feat], axis=1)
        xall_ref[...] = x_all
        pq_ref[...] = jnp.dot(x_all, uv_ref[...],
                              preferred_element_type=jnp.float32) + c_ref[...]

    const = lambda i: (0, 0)
    return pl.pallas_call(
        body,
        grid=(n // m,),
        in_specs=[
            pl.BlockSpec((m, n), lambda i: (i, 0)),
            pl.BlockSpec((n, 64), const),
            pl.BlockSpec((m, 32), lambda i: (i, 0)),
            pl.BlockSpec((m, 32), lambda i: (i, 0)),
            pl.BlockSpec((1, 32), const),
            pl.BlockSpec((1, 32), const),
            pl.BlockSpec((1, 32), const),
            pl.BlockSpec((96, 2), const),
            pl.BlockSpec((1, 1), const),
        ],
        out_specs=[
            pl.BlockSpec((m, 96), lambda i: (i, 0)),
            pl.BlockSpec((m, 2), lambda i: (i, 0)),
        ],
        out_shape=[
            jax.ShapeDtypeStruct((n, 96), jnp.float32),
            jax.ShapeDtypeStruct((n, 2), jnp.float32),
        ],
        compiler_params=pltpu.CompilerParams(
            dimension_semantics=("parallel",),
            vmem_limit_bytes=100 * 1024 * 1024,
        ),
    )(s_adj, r3, x1, ea, g_o2, b_s2o, b_s2, uvcat, chalf)


def _tc_forward(x, o_adj, s_adj, W_ogc1, b_ogc1, W_ogc2, b_ogc2, W_ogc1s,
                b_ogc1s, W_sgc1, b_sgc1, W_sgc2, b_sgc2, W_sgc1o, b_sgc1o,
                W_sgc2o, b_sgc2o, gate_o1, gate_s1, gate_o2, W_dec1, b_dec1,
                W_dec2, b_dec2):
    row = lambda v: v.reshape(1, -1)
    s_all = _proj(x, jnp.concatenate([W_ogc1, W_sgc1o, W_sgc1], axis=1))
    r2, ca, x2a = _pass1(o_adj, s_adj, s_all, row(gate_o1), row(b_ogc1),
                         row(b_sgc1o), row(b_sgc1), W_ogc1s, W_ogc2)
    r3, x1, ea = _pass2(o_adj, r2, ca, x2a, row(gate_s1), row(b_ogc1s),
                        row(b_ogc2), W_sgc2o, W_sgc2)
    nh = W_dec1.shape[0] // 2
    uv = W_dec1 @ W_dec2
    uvcat = jnp.concatenate([uv[:nh], uv[nh:]], axis=1)
    c = b_dec1 @ W_dec2 + b_dec2
    x_all, pq = _pass3(s_adj, r3, x1, ea, row(gate_o2), row(b_sgc2o),
                       row(b_sgc2), uvcat, (0.5 * c).reshape(1, 1))
    return x_all, pq


def _sc_linkpred(p, q, i0, i1):
    n, = p.shape
    b, = i0.shape
    info = plsc.get_sparse_core_info()
    nc, ns, lanes = info.num_cores, info.num_subcores, info.num_lanes
    nw = nc * ns
    bpw = b // nw
    mesh = plsc.VectorSubcoreMesh(core_axis_name="c", subcore_axis_name="s")

    @functools.partial(
        pl.kernel,
        mesh=mesh,
        out_type=jax.ShapeDtypeStruct((b,), jnp.float32),
        compiler_params=pltpu.CompilerParams(needs_layout_passes=False),
        scratch_types=[
            pltpu.VMEM((n,), jnp.float32),
            pltpu.VMEM((n,), jnp.float32),
            pltpu.VMEM((bpw,), jnp.int32),
            pltpu.VMEM((bpw,), jnp.int32),
            pltpu.VMEM((bpw,), jnp.float32),
        ],
    )
    def k(p_hbm, q_hbm, i0_hbm, i1_hbm, out_hbm, p_v, q_v, i0_v, i1_v, o_v):
        wid = lax.axis_index("s") * nc + lax.axis_index("c")
        base = wid * bpw
        pltpu.sync_copy(p_hbm, p_v)
        pltpu.sync_copy(q_hbm, q_v)
        pltpu.sync_copy(i0_hbm.at[pl.ds(base, bpw)], i0_v)
        pltpu.sync_copy(i1_hbm.at[pl.ds(base, bpw)], i1_v)

        def body(t, carry):
            sl = pl.ds(t * lanes, lanes)
            a = plsc.load_gather(p_v, [i0_v[sl]])
            bb = plsc.load_gather(q_v, [i1_v[sl]])
            o_v[sl] = a + bb
            return carry

        lax.fori_loop(0, bpw // lanes, body, 0)
        pltpu.sync_copy(o_v, out_hbm.at[pl.ds(base, bpw)])

    return k(p, q, i0, i1)


def kernel(x, o_adj, s_adj, idx, W_ogc1, b_ogc1, W_ogc2, b_ogc2, W_ogc1s,
           b_ogc1s, W_sgc1, b_sgc1, W_sgc2, b_sgc2, W_sgc1o, b_sgc1o, W_sgc2o,
           b_sgc2o, gate_o1, gate_s1, gate_o2, W_dec1, b_dec1, W_dec2, b_dec2):
    x_all, pq = _tc_forward(x, o_adj, s_adj, W_ogc1, b_ogc1, W_ogc2, b_ogc2,
                            W_ogc1s, b_ogc1s, W_sgc1, b_sgc1, W_sgc2, b_sgc2,
                            W_sgc1o, b_sgc1o, W_sgc2o, b_sgc2o, gate_o1,
                            gate_s1, gate_o2, W_dec1, b_dec1, W_dec2, b_dec2)
    o = _sc_linkpred(pq[:, 0], pq[:, 1], idx[0], idx[1])
    return o.reshape(-1, 1), x_all
